# baseline (speedup 1.0000x reference)
import functools

import jax
import jax.numpy as jnp
from jax import lax
from jax.experimental import pallas as pl
from jax.experimental.pallas import tpu as pltpu
from jax.experimental.pallas import tpu_sc as plsc

NC, NS, LANES = 2, 16, 16
NW = NC * NS
UNROLL = 16


def _add_rows(buf, pos_v, P):
    step = UNROLL * LANES

    def body(k, carry):
        b0 = k * step
        for u in range(UNROLL):
            o = b0 + u * LANES
            buf[pl.ds(o, LANES)] = buf[pl.ds(o, LANES)] + pos_v[pl.ds(o, LANES)]
        return carry

    lax.fori_loop(0, P // step, body, 0, unroll=False)


def kernel(encoded_tokens, pos_table):
    B, S, D = encoded_tokens.shape
    P = S * D
    n_rows = B // NW

    mesh = plsc.VectorSubcoreMesh(core_axis_name="c", subcore_axis_name="s")

    @functools.partial(
        pl.kernel,
        out_type=jax.ShapeDtypeStruct((B * P,), jnp.float32),
        mesh=mesh,
        scratch_types=[
            pltpu.VMEM((P,), jnp.float32),
            pltpu.VMEM((P,), jnp.float32),
        ],
    )
    def sc_add(tok_hbm, pos_hbm, out_hbm, pos_v, buf):
        wid = lax.axis_index("s") * NC + lax.axis_index("c")
        base = wid * (n_rows * P)
        pltpu.sync_copy(pos_hbm, pos_v)

        def row_body(i, carry):
            off = base + i * P
            pltpu.sync_copy(tok_hbm.at[pl.ds(off, P)], buf)
            _add_rows(buf, pos_v, P)
            pltpu.sync_copy(buf, out_hbm.at[pl.ds(off, P)])
            return carry

        lax.fori_loop(0, n_rows, row_body, 0, unroll=False)

    out = sc_add(encoded_tokens.reshape(B * P), pos_table.reshape(P))
    return out.reshape(B, S, D)
